# baseline (speedup 1.0000x reference)
import functools

import jax
import jax.numpy as jnp
from jax import lax
from jax.experimental import pallas as pl
from jax.experimental.pallas import tpu as pltpu
from jax.experimental.pallas import tpu_sc as plsc

_N, _S, _C = 16384, 50, 128
_NW = 32
_ROWS_W = _N // _NW
_CH = 8
_CHUNKS = _ROWS_W // _CH
_XW = _CH * _S
_BUFW = _CH * _S * _C


def _sc_onehot(x_hbm, out_hbm, xva, xvb, bufa, bufb, sema, semb):
    wid = lax.axis_index("s") * 2 + lax.axis_index("c")
    base = wid * _ROWS_W

    zero = jnp.zeros((16,), jnp.int32)
    one = jnp.ones((16,), jnp.int32)

    def zinit(i, carry):
        bufa[pl.ds(i * 16, 16)] = zero
        bufb[pl.ds(i * 16, 16)] = zero
        return carry

    lax.fori_loop(0, _BUFW // 16, zinit, 0)

    def scatter(buf, xv, val):
        def body(b, carry):
            vals = xv[pl.ds(b * 16, 16)]
            pos = lax.iota(jnp.int32, 16) + b * 16
            plsc.store_scatter(buf, [pos * _C + vals], val)
            return carry

        lax.fori_loop(0, _XW // 16, body, 0)

    def load_x(c, xv):
        row0 = base + c * _CH
        pltpu.sync_copy(x_hbm.at[pl.ds(row0 * _S, _XW)], xv)

    def out_copy(c, buf, sem):
        row0 = base + c * _CH
        return pltpu.make_async_copy(
            buf, out_hbm.at[pl.ds(row0 * _S * _C, _BUFW)], sem)

    bufs = ((bufa, xva, sema), (bufb, xvb, semb))

    for k, (buf, xv, sem) in enumerate(bufs):
        load_x(k, xv)
        scatter(buf, xv, one)
        out_copy(k, buf, sem).start()

    def pair(p, carry):
        for k, (buf, xv, sem) in enumerate(bufs):
            c = 2 * p + k
            out_copy(c - 2, buf, sem).wait()
            scatter(buf, xv, zero)
            load_x(c, xv)
            scatter(buf, xv, one)
            out_copy(c, buf, sem).start()
        return carry

    lax.fori_loop(1, _CHUNKS // 2, pair, 0)

    for k, (buf, xv, sem) in enumerate(bufs):
        out_copy(_CHUNKS - 2 + k, buf, sem).wait()


def kernel(x):
    n, s = x.shape
    xf = x.reshape(n * s)
    mesh = plsc.VectorSubcoreMesh(core_axis_name="c", subcore_axis_name="s")
    run = functools.partial(
        pl.kernel,
        mesh=mesh,
        compiler_params=pltpu.CompilerParams(needs_layout_passes=False),
        out_type=jax.ShapeDtypeStruct((n * s * _C,), jnp.int32),
        scratch_types=[
            pltpu.VMEM((_XW,), jnp.int32),
            pltpu.VMEM((_XW,), jnp.int32),
            pltpu.VMEM((_BUFW,), jnp.int32),
            pltpu.VMEM((_BUFW,), jnp.int32),
            pltpu.SemaphoreType.DMA,
            pltpu.SemaphoreType.DMA,
        ],
    )(_sc_onehot)
    return run(xf).reshape(n, s, _C)


# speedup vs baseline: 2.1183x; 2.1183x over previous
import jax
import jax.numpy as jnp
from jax import lax
from jax.experimental import pallas as pl
from jax.experimental.pallas import tpu as pltpu
from jax.experimental.pallas import tpu_sc as plsc

_N, _S, _C = 16384, 50, 128
_NW = 32
_ROWS_W = _N // _NW
_CH = 8
_CHUNKS = _ROWS_W // _CH
_XW = _CH * _S
_BUFW = _CH * _S * _C


def _sc_onehot(x_hbm, out_hbm, xva, xvb, bufa, bufb, sema, semb):
    wid = lax.axis_index("s") * 2 + lax.axis_index("c")
    base = wid * _ROWS_W

    zero = jnp.zeros((16,), jnp.int32)
    one = jnp.ones((16,), jnp.int32)

    def zinit(j, carry):
        for k in range(_C // 16):
            bufa[j, pl.ds(k * 16, 16)] = zero
            bufb[j, pl.ds(k * 16, 16)] = zero
        return carry

    lax.fori_loop(0, _XW, zinit, 0)

    def scatter(buf, xv, val):
        def body(b, carry):
            vals = xv[pl.ds(b * 16, 16)]
            pos = lax.iota(jnp.int32, 16) + b * 16
            plsc.store_scatter(buf, [pos, vals], val)
            return carry

        lax.fori_loop(0, _XW // 16, body, 0)

    def load_x(c, xv):
        row0 = base + c * _CH
        pltpu.sync_copy(x_hbm.at[pl.ds(row0 * _S, _XW)], xv)

    def out_copy(c, buf, sem):
        row0 = base + c * _CH
        return pltpu.make_async_copy(
            buf.reshape(_CH, _S, _C), out_hbm.at[pl.ds(row0, _CH)], sem)

    bufs = ((bufa, xva, sema), (bufb, xvb, semb))

    for k, (buf, xv, sem) in enumerate(bufs):
        load_x(k, xv)
        scatter(buf, xv, one)
        out_copy(k, buf, sem).start()

    def pair(p, carry):
        for k, (buf, xv, sem) in enumerate(bufs):
            c = 2 * p + k
            out_copy(c - 2, buf, sem).wait()
            scatter(buf, xv, zero)
            load_x(c, xv)
            scatter(buf, xv, one)
            out_copy(c, buf, sem).start()
        return carry

    lax.fori_loop(1, _CHUNKS // 2, pair, 0)

    for k, (buf, xv, sem) in enumerate(bufs):
        out_copy(_CHUNKS - 2 + k, buf, sem).wait()


def kernel(x):
    n, s = x.shape
    xf = x.reshape(n * s)
    mesh = plsc.VectorSubcoreMesh(core_axis_name="c", subcore_axis_name="s")
    run = pl.kernel(
        _sc_onehot,
        out_type=jax.ShapeDtypeStruct((n, s, _C), jnp.int32),
        mesh=mesh,
        compiler_params=pltpu.CompilerParams(needs_layout_passes=False),
        scratch_types=[
            pltpu.VMEM((_XW,), jnp.int32),
            pltpu.VMEM((_XW,), jnp.int32),
            pltpu.VMEM((_XW, _C), jnp.int32),
            pltpu.VMEM((_XW, _C), jnp.int32),
            pltpu.SemaphoreType.DMA,
            pltpu.SemaphoreType.DMA,
        ],
    )
    return run(xf)


# speedup vs baseline: 2.1535x; 1.0166x over previous
import jax
import jax.numpy as jnp
from jax import lax
from jax.experimental import pallas as pl
from jax.experimental.pallas import tpu as pltpu
from jax.experimental.pallas import tpu_sc as plsc

_N, _S, _C = 16384, 50, 128
_NW = 32
_ROWS_W = _N // _NW
_XWALL = _ROWS_W * _S
_CH = 8
_CHUNKS = _ROWS_W // _CH
_XW = _CH * _S
_BUFW = _XW * _C


def _sc_onehot(x_hbm, out_hbm, xall, bufa, bufb, sema, semb):
    wid = lax.axis_index("s") * 2 + lax.axis_index("c")
    base = wid * _ROWS_W

    zero = jnp.zeros((16,), jnp.int32)
    one = jnp.ones((16,), jnp.int32)

    pltpu.sync_copy(x_hbm.at[pl.ds(wid * _XWALL, _XWALL)], xall)

    def zinit(j, carry):
        for k in range(_C // 16):
            bufa[j, pl.ds(k * 16, 16)] = zero
            bufb[j, pl.ds(k * 16, 16)] = zero
        return carry

    lax.fori_loop(0, _XW, zinit, 0)

    def scatter(buf, c, val):
        x0 = c * _XW

        def body(b, carry):
            vals = xall[pl.ds(x0 + b * 16, 16)]
            pos = lax.iota(jnp.int32, 16) + b * 16
            plsc.store_scatter(buf, [pos, vals], val)
            return carry

        lax.fori_loop(0, _XW // 16, body, 0)

    def out_copy(c, buf, sem):
        row0 = base + c * _CH
        return pltpu.make_async_copy(
            buf.reshape(_CH, _S, _C), out_hbm.at[pl.ds(row0, _CH)], sem)

    bufs = ((bufa, sema), (bufb, semb))

    for k, (buf, sem) in enumerate(bufs):
        scatter(buf, k, one)
        out_copy(k, buf, sem).start()

    def pair(p, carry):
        for k, (buf, sem) in enumerate(bufs):
            c = 2 * p + k
            out_copy(c - 2, buf, sem).wait()
            scatter(buf, c - 2, zero)
            scatter(buf, c, one)
            out_copy(c, buf, sem).start()
        return carry

    lax.fori_loop(1, _CHUNKS // 2, pair, 0)

    for k, (buf, sem) in enumerate(bufs):
        out_copy(_CHUNKS - 2 + k, buf, sem).wait()


def kernel(x):
    n, s = x.shape
    xf = x.reshape(n * s)
    mesh = plsc.VectorSubcoreMesh(core_axis_name="c", subcore_axis_name="s")
    run = pl.kernel(
        _sc_onehot,
        out_type=jax.ShapeDtypeStruct((n, s, _C), jnp.int32),
        mesh=mesh,
        compiler_params=pltpu.CompilerParams(needs_layout_passes=False),
        scratch_types=[
            pltpu.VMEM((_XWALL,), jnp.int32),
            pltpu.VMEM((_XW, _C), jnp.int32),
            pltpu.VMEM((_XW, _C), jnp.int32),
            pltpu.SemaphoreType.DMA,
            pltpu.SemaphoreType.DMA,
        ],
    )
    return run(xf)
